# Initial kernel scaffold; baseline (speedup 1.0000x reference)
import functools

import jax
import jax.numpy as jnp
from jax import lax
from jax.experimental import pallas as pl
from jax.experimental.pallas import tpu as pltpu
from jax.experimental.pallas import tpu_sc as plsc

BATCH = 16384
MAX_LENGTH = 20
EMBED_DIM = 16

NUM_CORES = 2
NUM_SUBCORES = 16
NW = NUM_CORES * NUM_SUBCORES
B_PER_W = BATCH // NW
T_PER_W = B_PER_W * MAX_LENGTH
IDX_W = 128
IDX_ROWS_PER_W = T_PER_W // IDX_W
CHUNK_B = 128
CHUNK_T = CHUNK_B * MAX_LENGTH
IDX_ROWS_PER_CHUNK = CHUNK_T // IDX_W
N_CHUNK = B_PER_W // CHUNK_B

_mesh = plsc.VectorSubcoreMesh(core_axis_name="c", subcore_axis_name="s")


@functools.partial(
    pl.kernel,
    out_type=jax.ShapeDtypeStruct((BATCH, EMBED_DIM), jnp.float32),
    mesh=_mesh,
    scratch_types=[
        pltpu.VMEM((IDX_ROWS_PER_W, IDX_W), jnp.int32),
        pltpu.VMEM((2, CHUNK_T, EMBED_DIM), jnp.float32),
        pltpu.VMEM((CHUNK_B, EMBED_DIM), jnp.float32),
        pltpu.VMEM((1, EMBED_DIM), jnp.float32),
        pltpu.SemaphoreType.DMA,
        pltpu.SemaphoreType.DMA,
    ],
)
def _sc_pool(ids_hbm, table_hbm, out_hbm, idx_v, rows_v, out_v, row0_v,
             sem_a, sem_b):
    wid = lax.axis_index("s") * NUM_CORES + lax.axis_index("c")
    sems = (sem_a, sem_b)

    pltpu.sync_copy(ids_hbm.at[pl.ds(wid * IDX_ROWS_PER_W, IDX_ROWS_PER_W)],
                    idx_v)
    pltpu.sync_copy(table_hbm.at[pl.ds(0, 1)], row0_v)

    def fire(c, buf):
        cps = []
        for j in range(IDX_ROWS_PER_CHUNK):
            cps.append(pltpu.async_copy(
                table_hbm.at[idx_v.at[c * IDX_ROWS_PER_CHUNK + j]],
                rows_v.at[buf].at[pl.ds(j * IDX_W, IDX_W)],
                sems[buf]))
        return cps

    lane = lax.iota(jnp.int32, 16)

    def process(c, buf, row0):
        def body(b, _):
            tb = c * CHUNK_T + b * MAX_LENGTH
            t1 = tb + lane
            v1 = plsc.load_gather(idx_v, [t1 >> 7, t1 & 127])
            t2 = jnp.where(lane < 4, tb + 16 + lane, tb)
            v2 = plsc.load_gather(idx_v, [t2 >> 7, t2 & 127])
            nz1 = jnp.where(v1 != 0, 1.0, 0.0)
            nz2 = jnp.where((lane < 4) & (v2 != 0), 1.0, 0.0)
            cnt = jnp.sum(nz1) + jnp.sum(nz2)
            cnt_v = jnp.full((16,), cnt, jnp.float32)
            npad_v = 20.0 - cnt_v
            inv = 1.0 / jnp.maximum(cnt_v, 1.0)
            tl = b * MAX_LENGTH
            acc = rows_v[buf, tl, :]
            for l in range(1, MAX_LENGTH):
                acc = acc + rows_v[buf, tl + l, :]
            out_v[b, :] = (acc - npad_v * row0) * inv
            return 0

        lax.fori_loop(0, CHUNK_B, body, 0)

    cps = [None, None]
    cps[0] = fire(0, 0)
    row0 = None
    for c in range(N_CHUNK):
        buf = c & 1
        if c + 1 < N_CHUNK:
            cps[1 - buf] = fire(c + 1, 1 - buf)
        for cp in cps[buf]:
            cp.wait()
        if row0 is None:
            row0 = row0_v[0, :]
        process(c, buf, row0)
        pltpu.sync_copy(out_v,
                        out_hbm.at[pl.ds(wid * B_PER_W + c * CHUNK_B,
                                         CHUNK_B)])


def kernel(token_ids, table):
    ids2d = token_ids.reshape(BATCH * MAX_LENGTH // IDX_W, IDX_W)
    return _sc_pool(ids2d, table)


# speedup vs baseline: 1.0230x; 1.0230x over previous
import functools

import jax
import jax.numpy as jnp
from jax import lax
from jax.experimental import pallas as pl
from jax.experimental.pallas import tpu as pltpu
from jax.experimental.pallas import tpu_sc as plsc

BATCH = 16384
MAX_LENGTH = 20
EMBED_DIM = 16

NUM_CORES = 2
NUM_SUBCORES = 16
NW = NUM_CORES * NUM_SUBCORES
B_PER_W = BATCH // NW
T_PER_W = B_PER_W * MAX_LENGTH
IDX_W = 128
IDX_ROWS_PER_W = T_PER_W // IDX_W
CHUNK_B = 128
CHUNK_T = CHUNK_B * MAX_LENGTH
IDX_ROWS_PER_CHUNK = CHUNK_T // IDX_W
N_CHUNK = B_PER_W // CHUNK_B

_mesh = plsc.VectorSubcoreMesh(core_axis_name="c", subcore_axis_name="s")


@functools.partial(
    pl.kernel,
    out_type=jax.ShapeDtypeStruct((BATCH, EMBED_DIM), jnp.float32),
    mesh=_mesh,
    scratch_types=[
        pltpu.VMEM((T_PER_W,), jnp.int32),
        pltpu.VMEM((2, CHUNK_T, EMBED_DIM), jnp.float32),
        pltpu.VMEM((CHUNK_B, EMBED_DIM), jnp.float32),
        pltpu.VMEM((1, EMBED_DIM), jnp.float32),
        pltpu.SemaphoreType.DMA,
        pltpu.SemaphoreType.DMA,
    ],
    compiler_params=pltpu.CompilerParams(use_tc_tiling_on_sc=False),
)
def _sc_pool(ids_hbm, table_hbm, out_hbm, idx_v, rows_v, out_v, row0_v,
             sem_a, sem_b):
    wid = lax.axis_index("s") * NUM_CORES + lax.axis_index("c")
    sems = (sem_a, sem_b)
    pltpu.sync_copy(ids_hbm.at[pl.ds(wid * T_PER_W, T_PER_W)], idx_v)
    pltpu.sync_copy(table_hbm.at[pl.ds(0, 1)], row0_v)

    def fire(c, buf):
        cps = []
        for j in range(IDX_ROWS_PER_CHUNK):
            cps.append(pltpu.async_copy(
                table_hbm.at[idx_v.at[pl.ds(
                    (c * IDX_ROWS_PER_CHUNK + j) * IDX_W, IDX_W)]],
                rows_v.at[buf].at[pl.ds(j * IDX_W, IDX_W)],
                sems[buf]))
        return cps

    lane = lax.iota(jnp.int32, 16)

    def process(c, buf, row0):
        def body(b, _):
            tb = c * CHUNK_T + b * MAX_LENGTH
            v1 = idx_v[pl.ds(tb, 16)]
            v2 = idx_v[pl.ds(tb + 4, 16)]
            m = (jnp.where(v1 != 0, 1, 0)
                 + jnp.where((lane >= 12) & (v2 != 0), 1, 0))
            parts = [m[i] for i in range(16)]
            while len(parts) > 1:
                parts = [parts[i] + parts[i + 1]
                         for i in range(0, len(parts), 2)]
            cnt_v = jnp.full((16,), parts[0], jnp.int32).astype(jnp.float32)
            npad_v = 20.0 - cnt_v
            inv = 1.0 / jnp.maximum(cnt_v, 1.0)
            tl = b * MAX_LENGTH
            acc = rows_v[buf, tl, :]
            for l in range(1, MAX_LENGTH):
                acc = acc + rows_v[buf, tl + l, :]
            out_v[b, :] = (acc - npad_v * row0) * inv
            return 0

        lax.fori_loop(0, CHUNK_B, body, 0)

    cps = [None, None]
    cps[0] = fire(0, 0)
    row0 = None
    for c in range(N_CHUNK):
        buf = c & 1
        if c + 1 < N_CHUNK:
            cps[1 - buf] = fire(c + 1, 1 - buf)
        for cp in cps[buf]:
            cp.wait()
        if row0 is None:
            row0 = row0_v[0, :]
        process(c, buf, row0)
        pltpu.sync_copy(out_v,
                        out_hbm.at[pl.ds(wid * B_PER_W + c * CHUNK_B,
                                         CHUNK_B)])


def kernel(token_ids, table):
    return _sc_pool(token_ids.reshape(BATCH * MAX_LENGTH), table)


# speedup vs baseline: 1.5254x; 1.4911x over previous
import functools

import jax
import jax.numpy as jnp
from jax import lax
from jax.experimental import pallas as pl
from jax.experimental.pallas import tpu as pltpu
from jax.experimental.pallas import tpu_sc as plsc

BATCH = 16384
MAX_LENGTH = 20
EMBED_DIM = 16

NUM_CORES = 2
NUM_SUBCORES = 16
NW = NUM_CORES * NUM_SUBCORES
B_PER_W = BATCH // NW
T_PER_W = B_PER_W * MAX_LENGTH
IDX_W = 128
IDX_ROWS_PER_W = T_PER_W // IDX_W
CHUNK_B = 128
CHUNK_T = CHUNK_B * MAX_LENGTH
IDX_ROWS_PER_CHUNK = CHUNK_T // IDX_W
N_CHUNK = B_PER_W // CHUNK_B

_mesh = plsc.VectorSubcoreMesh(core_axis_name="c", subcore_axis_name="s")

VOCAB = 1000000
TR_W = 8192
TR_OUT_R = TR_W * EMBED_DIM // 128


def _tr_body(in_ref, out_ref, scr_ref):
    for a in range(TR_W // 256):
        sl = pl.ds(a * 256, 256)
        scr_ref[sl, :] = in_ref[:, sl].T
        parts = [scr_ref[a * 256 + j:(a + 1) * 256:8, :] for j in range(8)]
        out_ref[pl.ds(a * 32, 32), :] = jnp.concatenate(parts, axis=1)


def _tc_relayout(table_t):
    return pl.pallas_call(
        _tr_body,
        grid=(pl.cdiv(VOCAB, TR_W),),
        in_specs=[pl.BlockSpec((EMBED_DIM, TR_W), lambda i: (0, i))],
        out_specs=pl.BlockSpec((TR_OUT_R, 128), lambda i: (i, 0)),
        out_shape=jax.ShapeDtypeStruct((VOCAB * EMBED_DIM // 128, 128),
                                       jnp.float32),
        scratch_shapes=[pltpu.VMEM((TR_W, EMBED_DIM), jnp.float32)],
    )(table_t)


@functools.partial(
    pl.kernel,
    out_type=jax.ShapeDtypeStruct((BATCH, EMBED_DIM), jnp.float32),
    mesh=_mesh,
    scratch_types=[
        pltpu.VMEM((T_PER_W,), jnp.int32),
        pltpu.VMEM((2, CHUNK_T, EMBED_DIM), jnp.float32),
        pltpu.VMEM((CHUNK_B, EMBED_DIM), jnp.float32),
        pltpu.VMEM((1, EMBED_DIM), jnp.float32),
        pltpu.SemaphoreType.DMA,
        pltpu.SemaphoreType.DMA,
    ],
    compiler_params=pltpu.CompilerParams(use_tc_tiling_on_sc=False),
)
def _sc_pool(ids_hbm, table_hbm, out_hbm, idx_v, rows_v, out_v, row0_v,
             sem_a, sem_b):
    wid = lax.axis_index("s") * NUM_CORES + lax.axis_index("c")
    sems = (sem_a, sem_b)
    pltpu.sync_copy(ids_hbm.at[pl.ds(wid * T_PER_W, T_PER_W)], idx_v)
    pltpu.sync_copy(table_hbm.at[pl.ds(0, 1)], row0_v)

    def fire(c, buf):
        cps = []
        for j in range(IDX_ROWS_PER_CHUNK):
            cps.append(pltpu.async_copy(
                table_hbm.at[idx_v.at[pl.ds(
                    (c * IDX_ROWS_PER_CHUNK + j) * IDX_W, IDX_W)]],
                rows_v.at[buf].at[pl.ds(j * IDX_W, IDX_W)],
                sems[buf]))
        return cps

    lane = lax.iota(jnp.int32, 16)

    def process(c, buf, row0):
        def body(b, _):
            tb = c * CHUNK_T + b * MAX_LENGTH
            v1 = idx_v[pl.ds(tb, 16)]
            v2 = idx_v[pl.ds(tb + 4, 16)]
            m = (jnp.where(v1 != 0, 1, 0)
                 + jnp.where((lane >= 12) & (v2 != 0), 1, 0))
            parts = [m[i] for i in range(16)]
            while len(parts) > 1:
                parts = [parts[i] + parts[i + 1]
                         for i in range(0, len(parts), 2)]
            cnt_v = jnp.full((16,), parts[0], jnp.int32).astype(jnp.float32)
            npad_v = 20.0 - cnt_v
            inv = 1.0 / jnp.maximum(cnt_v, 1.0)
            tl = b * MAX_LENGTH
            acc = rows_v[buf, tl, :]
            for l in range(1, MAX_LENGTH):
                acc = acc + rows_v[buf, tl + l, :]
            out_v[b, :] = (acc - npad_v * row0) * inv
            return 0

        lax.fori_loop(0, CHUNK_B, body, 0)

    cps = [None, None]
    cps[0] = fire(0, 0)
    row0 = None
    for c in range(N_CHUNK):
        buf = c & 1
        if c + 1 < N_CHUNK:
            cps[1 - buf] = fire(c + 1, 1 - buf)
        for cp in cps[buf]:
            cp.wait()
        if row0 is None:
            row0 = row0_v[0, :]
        process(c, buf, row0)
        pltpu.sync_copy(out_v,
                        out_hbm.at[pl.ds(wid * B_PER_W + c * CHUNK_B,
                                         CHUNK_B)])


def kernel(token_ids, table):
    table_rm = _tc_relayout(table.T).reshape(VOCAB, EMBED_DIM)
    return _sc_pool(token_ids.reshape(BATCH * MAX_LENGTH), table_rm)
